# baseline (speedup 1.0000x reference)
import functools

import jax
import jax.numpy as jnp
from jax import lax
from jax.experimental import pallas as pl
from jax.experimental.pallas import tpu as pltpu
from jax.experimental.pallas import tpu_sc as plsc

_VOCAB = 32
_NEG = -1e9
_POS = 1e9


def kernel(hidden, next_ids):
    del hidden
    B, T = next_ids.shape
    N = B * T
    ids = next_ids.astype(jnp.int32)
    NW_CHUNKS = 4

    info = plsc.get_sparse_core_info()
    NC, NS, L = info.num_cores, info.num_subcores, info.num_lanes
    NW = NC * NS
    nper = N // NW
    t_chunk = nper // NW_CHUNKS
    g_per_chunk = t_chunk // L

    mesh = plsc.VectorSubcoreMesh(core_axis_name="c", subcore_axis_name="s")

    @functools.partial(
        pl.kernel,
        mesh=mesh,
        out_type=jax.ShapeDtypeStruct((B, _VOCAB, T), jnp.float32),
        scratch_types=[
            pltpu.VMEM((nper,), jnp.int32),
            pltpu.VMEM((_VOCAB, nper), jnp.float32),
            pltpu.SemaphoreType.DMA,
            pltpu.SemaphoreType.DMA,
        ],
        compiler_params=pltpu.CompilerParams(needs_layout_passes=False),
    )
    def sc_onehot(ids_hbm, out_hbm, idx_v, buf, sem_ids, sem_out):
        wid = lax.axis_index("s") * NC + lax.axis_index("c")
        base = wid * nper
        b = base // T
        t0 = base % T

        id_cp = pltpu.async_copy(ids_hbm.at[b, pl.ds(t0, nper)], idx_v,
                                 sem_ids)

        neg = jnp.full((L,), _NEG, jnp.float32)
        lane = lax.iota(jnp.int32, L)
        pos = jnp.full((L,), _POS, jnp.float32)

        def fill(k):
            def body(i, c):
                col = k * t_chunk + i * L
                for v in range(_VOCAB):
                    buf[v, pl.ds(col, L)] = neg
                return c

            lax.fori_loop(0, t_chunk // L, body, 0)

        def scat(k):
            def body(g, c):
                tok = g * L
                v = idx_v[pl.ds(tok, L)]
                tgt = (v + 1) & (_VOCAB - 1)
                plsc.store_scatter(buf, [tgt, lane + tok], pos)
                return c

            lax.fori_loop(k * g_per_chunk, (k + 1) * g_per_chunk, body, 0)

        out_cps = []

        def ship(k):
            scat(k)
            out_cps.append(
                pltpu.async_copy(
                    buf.at[:, pl.ds(k * t_chunk, t_chunk)],
                    out_hbm.at[b, :, pl.ds(t0 + k * t_chunk, t_chunk)],
                    sem_out,
                )
            )

        fill(0)
        id_cp.wait()
        for k in range(1, NW_CHUNKS):
            fill(k)
            ship(k - 1)
        ship(NW_CHUNKS - 1)
        for cp in out_cps:
            cp.wait()

    out_t = sc_onehot(ids)
    return jnp.swapaxes(out_t, 1, 2)


# speedup vs baseline: 1.0080x; 1.0080x over previous
import functools

import jax
import jax.numpy as jnp
from jax import lax
from jax.experimental import pallas as pl
from jax.experimental.pallas import tpu as pltpu
from jax.experimental.pallas import tpu_sc as plsc

_VOCAB = 32
_NEG = -1e9
_POS = 1e9


def kernel(hidden, next_ids):
    del hidden
    B, T = next_ids.shape
    N = B * T
    ids = next_ids.reshape(N).astype(jnp.int32)
    NW_CHUNKS = 4

    info = plsc.get_sparse_core_info()
    NC, NS, L = info.num_cores, info.num_subcores, info.num_lanes
    NW = NC * NS
    nper = N // NW
    t_chunk = nper // NW_CHUNKS
    g_per_chunk = t_chunk // L

    mesh = plsc.VectorSubcoreMesh(core_axis_name="c", subcore_axis_name="s")

    @functools.partial(
        pl.kernel,
        mesh=mesh,
        out_type=jax.ShapeDtypeStruct((B, _VOCAB, T), jnp.float32),
        scratch_types=[
            pltpu.VMEM((nper,), jnp.int32),
            pltpu.VMEM((_VOCAB, nper), jnp.float32),
            pltpu.SemaphoreType.DMA,
            pltpu.SemaphoreType.DMA,
        ],
        compiler_params=pltpu.CompilerParams(
            needs_layout_passes=False, skip_device_barrier=True
        ),
    )
    def sc_onehot(ids_hbm, out_hbm, idx_v, buf, sem_ids, sem_out):
        wid = lax.axis_index("s") * NC + lax.axis_index("c")
        base = wid * nper
        b = base // T
        t0 = base % T

        id_cp = pltpu.async_copy(ids_hbm.at[pl.ds(base, nper)], idx_v,
                                 sem_ids)

        neg = jnp.full((L,), _NEG, jnp.float32)
        lane = lax.iota(jnp.int32, L)
        pos = jnp.full((L,), _POS, jnp.float32)

        def fill(k):
            def body(i, c):
                col = k * t_chunk + i * L
                for v in range(_VOCAB):
                    buf[v, pl.ds(col, L)] = neg
                return c

            lax.fori_loop(0, t_chunk // L, body, 0)

        def scat(k):
            def body(g, c):
                tok = g * L
                v = idx_v[pl.ds(tok, L)]
                tgt = (v + 1) & (_VOCAB - 1)
                plsc.store_scatter(buf, [tgt, lane + tok], pos)
                return c

            lax.fori_loop(k * g_per_chunk, (k + 1) * g_per_chunk, body, 0)

        out_cps = []

        def ship(k):
            scat(k)
            out_cps.append(
                pltpu.async_copy(
                    buf.at[:, pl.ds(k * t_chunk, t_chunk)],
                    out_hbm.at[b, :, pl.ds(t0 + k * t_chunk, t_chunk)],
                    sem_out,
                )
            )

        fill(0)
        id_cp.wait()
        for k in range(1, NW_CHUNKS):
            fill(k)
            ship(k - 1)
        ship(NW_CHUNKS - 1)
        for cp in out_cps:
            cp.wait()

    out_t = sc_onehot(ids)
    return jnp.swapaxes(out_t, 1, 2)


# speedup vs baseline: 1.0336x; 1.0254x over previous
import functools

import jax
import jax.numpy as jnp
from jax import lax
from jax.experimental import pallas as pl
from jax.experimental.pallas import tpu as pltpu
from jax.experimental.pallas import tpu_sc as plsc

_VOCAB = 32
_NEG = -1e9
_POS = 1e9


def kernel(hidden, next_ids):
    del hidden
    B, T = next_ids.shape
    N = B * T
    ids = next_ids.reshape(N).astype(jnp.int32)

    info = plsc.get_sparse_core_info()
    NC, NS, L = info.num_cores, info.num_subcores, info.num_lanes
    NW = NC * NS
    nper = N // NW

    mesh = plsc.VectorSubcoreMesh(core_axis_name="c", subcore_axis_name="s")

    @functools.partial(
        pl.kernel,
        mesh=mesh,
        out_type=jax.ShapeDtypeStruct((B, _VOCAB, T), jnp.float32),
        scratch_types=[
            pltpu.VMEM((nper,), jnp.int32),
            pltpu.VMEM((_VOCAB, nper), jnp.float32),
            pltpu.SemaphoreType.DMA,
            pltpu.SemaphoreType.DMA,
        ],
        compiler_params=pltpu.CompilerParams(needs_layout_passes=False),
    )
    def sc_onehot(ids_hbm, out_hbm, idx_v, buf, sem_ids, sem_out):
        wid = lax.axis_index("s") * NC + lax.axis_index("c")
        base = wid * nper
        b = base // T
        t0 = base % T

        id_cp = pltpu.async_copy(ids_hbm.at[pl.ds(base, nper)], idx_v,
                                 sem_ids)

        neg = jnp.full((L,), _NEG, jnp.float32)
        lane = lax.iota(jnp.int32, L)
        pos = jnp.full((L,), _POS, jnp.float32)

        def fill_body(i, c):
            v = i // (nper // (8 * L))
            j = i % (nper // (8 * L))
            for u in range(8):
                buf[v, pl.ds((j * 8 + u) * L, L)] = neg
            return c

        lax.fori_loop(0, _VOCAB * nper // (8 * L), fill_body, 0)
        id_cp.wait()

        def scat_body(g, c):
            tok = g * L
            v = idx_v[pl.ds(tok, L)]
            tgt = (v + 1) & (_VOCAB - 1)
            plsc.store_scatter(buf, [tgt, lane + tok], pos)
            return c

        lax.fori_loop(0, nper // L, scat_body, 0)

        pltpu.async_copy(
            buf, out_hbm.at[b, :, pl.ds(t0, nper)], sem_out
        ).wait()

    out_t = sc_onehot(ids)
    return jnp.swapaxes(out_t, 1, 2)


# speedup vs baseline: 1.0434x; 1.0095x over previous
import functools

import jax
import jax.numpy as jnp
from jax import lax
from jax.experimental import pallas as pl
from jax.experimental.pallas import tpu as pltpu
from jax.experimental.pallas import tpu_sc as plsc

_VOCAB = 32
_NEG = -1e9
_POS = 1e9


def kernel(hidden, next_ids):
    del hidden
    B, T = next_ids.shape
    N = B * T
    ids = next_ids.reshape(N).astype(jnp.int32)

    info = plsc.get_sparse_core_info()
    NC, NS, L = info.num_cores, info.num_subcores, info.num_lanes
    NW = NC * NS
    nper = N // NW

    mesh = plsc.VectorSubcoreMesh(core_axis_name="c", subcore_axis_name="s")

    @functools.partial(
        pl.kernel,
        mesh=mesh,
        out_type=jax.ShapeDtypeStruct((B, _VOCAB, T), jnp.float32),
        scratch_types=[
            pltpu.VMEM((nper,), jnp.int32),
            pltpu.VMEM((_VOCAB, nper), jnp.float32),
            pltpu.SemaphoreType.DMA,
            pltpu.SemaphoreType.DMA,
        ],
        compiler_params=pltpu.CompilerParams(needs_layout_passes=False),
    )
    def sc_onehot(ids_hbm, out_hbm, idx_v, buf, sem_ids, sem_out):
        wid = lax.axis_index("s") * NC + lax.axis_index("c")
        base = wid * nper
        b = base // T
        t0 = base % T

        id_cp = pltpu.async_copy(ids_hbm.at[pl.ds(base, nper)], idx_v,
                                 sem_ids)

        neg = jnp.full((L,), _NEG, jnp.float32)
        lane = lax.iota(jnp.int32, L)
        pos = jnp.full((L,), _POS, jnp.float32)

        half = nper // 2
        j_per_half = half // (8 * L)

        def fill_body(i, c):
            v = i // j_per_half
            j = i % j_per_half
            col = c + (j * 8) * L
            for u in range(8):
                buf[v, pl.ds(col + u * L, L)] = neg
            return c

        def scat_body(g, c):
            tok = g * L
            v = idx_v[pl.ds(tok, L)]
            tgt = (v + 1) & (_VOCAB - 1)
            plsc.store_scatter(buf, [tgt, lane + tok], pos)
            return c

        cps = []
        for h in range(2):
            lax.fori_loop(0, _VOCAB * j_per_half, fill_body, h * half)
            if h == 0:
                id_cp.wait()
            lax.fori_loop(h * (half // L), (h + 1) * (half // L),
                          scat_body, 0)
            cps.append(
                pltpu.async_copy(
                    buf.at[:, pl.ds(h * half, half)],
                    out_hbm.at[b, :, pl.ds(t0 + h * half, half)],
                    sem_out,
                )
            )
        cps[0].wait()
        cps[1].wait()

    out_t = sc_onehot(ids)
    return jnp.swapaxes(out_t, 1, 2)
